# Initial kernel scaffold; baseline (speedup 1.0000x reference)
import functools

import jax
import jax.numpy as jnp
from jax import lax
from jax.experimental import pallas as pl
from jax.experimental.pallas import tpu as pltpu
from jax.experimental.pallas import tpu_sc as plsc

N = 10000
E = 320000
D = 128

NC = 2
NS = 16
EDGES_PER_TILE = E // (NC * NS)
CHUNK = 400
NCHUNKS = EDGES_PER_TILE // CHUNK
ROWS_PER_TILE = N // NS


def _sc_body(x_hbm, src_hbm, dst_hbm, w_hbm, out_hbm,
             idx_src, idx_dst, wbuf, rows, agg, sem):
    c = lax.axis_index("c")
    s = lax.axis_index("s")

    def _zrow(i, _):
        for f in range(D // 16):
            rows[i, pl.ds(f * 16, 16)] = jnp.zeros((16,), jnp.float32)
        return 0
    lax.fori_loop(0, CHUNK, _zrow, 0)
    r0 = s * ROWS_PER_TILE
    pltpu.sync_copy(rows, agg.at[pl.ds(r0, CHUNK)])
    pltpu.sync_copy(rows.at[pl.ds(0, ROWS_PER_TILE - CHUNK)],
                    agg.at[pl.ds(r0 + CHUNK, ROWS_PER_TILE - CHUNK)])
    plsc.subcore_barrier()

    base = (c * NS + s) * EDGES_PER_TILE

    def _chunk(k, _):
        off = base + k * CHUNK
        pltpu.sync_copy(src_hbm.at[pl.ds(off, CHUNK)], idx_src)
        pltpu.sync_copy(dst_hbm.at[pl.ds(off, CHUNK)], idx_dst)
        pltpu.sync_copy(w_hbm.at[pl.ds(off, CHUNK)], wbuf)
        pltpu.async_copy(x_hbm.at[idx_src], rows, sem).wait()

        def _scale(i, _):
            wv = plsc.load_gather(wbuf, [jnp.full((16,), i, jnp.int32)])
            for f in range(D // 16):
                sl = pl.ds(f * 16, 16)
                rows[i, sl] = rows[i, sl] * wv
            return 0
        lax.fori_loop(0, CHUNK, _scale, 0)

        pltpu.sync_copy(rows, agg.at[idx_dst], add=True)
        return 0
    lax.fori_loop(0, NCHUNKS, _chunk, 0)
    plsc.subcore_barrier()

    pltpu.sync_copy(agg.at[pl.ds(r0, CHUNK)], rows)
    pltpu.sync_copy(rows, out_hbm.at[c, pl.ds(r0, CHUNK)])
    pltpu.sync_copy(agg.at[pl.ds(r0 + CHUNK, ROWS_PER_TILE - CHUNK)],
                    rows.at[pl.ds(0, ROWS_PER_TILE - CHUNK)])
    pltpu.sync_copy(rows.at[pl.ds(0, ROWS_PER_TILE - CHUNK)],
                    out_hbm.at[c, pl.ds(r0 + CHUNK, ROWS_PER_TILE - CHUNK)])


_sc_agg = functools.partial(
    pl.kernel,
    out_type=jax.ShapeDtypeStruct((NC, N, D), jnp.float32),
    mesh=plsc.VectorSubcoreMesh(core_axis_name="c", subcore_axis_name="s"),
    scratch_types=[
        pltpu.VMEM((CHUNK,), jnp.int32),
        pltpu.VMEM((CHUNK,), jnp.int32),
        pltpu.VMEM((CHUNK,), jnp.float32),
        pltpu.VMEM((CHUNK, D), jnp.float32),
        pltpu.VMEM_SHARED((N, D), jnp.float32),
        pltpu.SemaphoreType.DMA,
    ],
)(_sc_body)


def _tc_body(scal_ref, p_ref, h_ref, w_ref, o_ref):
    alpha = scal_ref[0]
    beta = scal_ref[1]
    left = (1.0 - alpha) * (p_ref[0] + p_ref[1]) + alpha * h_ref[...]
    o_ref[...] = (1.0 - beta) * left + beta * jnp.dot(
        left, w_ref[...], preferred_element_type=jnp.float32)


_ROWS_BLK = 1000

_tc_finish = pl.pallas_call(
    _tc_body,
    grid=(N // _ROWS_BLK,),
    in_specs=[
        pl.BlockSpec(memory_space=pltpu.SMEM),
        pl.BlockSpec((NC, _ROWS_BLK, D), lambda i: (0, i, 0)),
        pl.BlockSpec((_ROWS_BLK, D), lambda i: (i, 0)),
        pl.BlockSpec((D, D), lambda i: (0, 0)),
    ],
    out_specs=pl.BlockSpec((_ROWS_BLK, D), lambda i: (i, 0)),
    out_shape=jax.ShapeDtypeStruct((N, D), jnp.float32),
)


def kernel(x, h0, W, adj_values, adj_edge_index, alpha, beta):
    dst = adj_edge_index[0]
    src = adj_edge_index[1]
    partials = _sc_agg(x, src, dst, adj_values)
    scal = jnp.stack([jnp.asarray(alpha, jnp.float32),
                      jnp.asarray(beta, jnp.float32)])
    return _tc_finish(scal, partials, h0, W)


# speedup vs baseline: 4.5245x; 4.5245x over previous
import functools

import jax
import jax.numpy as jnp
from jax import lax
from jax.experimental import pallas as pl
from jax.experimental.pallas import tpu as pltpu
from jax.experimental.pallas import tpu_sc as plsc

N = 10000
E = 320000
D = 128

NC = 2
NS = 16
EDGES_PER_TILE = E // (NC * NS)
CHUNK = 80
NCHUNKS = EDGES_PER_TILE // CHUNK
N_PAD = 10240
ROWS_PER_TILE = N_PAD // NS


def _sc_body(x_hbm, src_hbm, dst_hbm, w_hbm, out_hbm,
             idx_src, idx_dst, wbuf, rows, agg, sem):
    c = lax.axis_index("c")
    s = lax.axis_index("s")

    def _zrow(i, _):
        for f in range(D // 16):
            rows[i, pl.ds(f * 16, 16)] = jnp.zeros((16,), jnp.float32)
        return 0
    lax.fori_loop(0, CHUNK, _zrow, 0)
    r0 = s * ROWS_PER_TILE
    segs = []
    o = 0
    while o < ROWS_PER_TILE:
        segs.append((o, min(CHUNK, ROWS_PER_TILE - o)))
        o += CHUNK
    for o, n in segs:
        pltpu.sync_copy(rows.at[pl.ds(0, n)], agg.at[pl.ds(r0 + o, n)])
    plsc.subcore_barrier()

    base = (c * NS + s) * EDGES_PER_TILE

    def _chunk(k, _):
        off = base + k * CHUNK
        pltpu.sync_copy(src_hbm.at[pl.ds(off, CHUNK)], idx_src)
        pltpu.sync_copy(dst_hbm.at[pl.ds(off, CHUNK)], idx_dst)
        pltpu.sync_copy(w_hbm.at[pl.ds(off, CHUNK)], wbuf)
        pltpu.async_copy(x_hbm.at[idx_src], rows, sem).wait()

        def _scale(g, _):
            wv16 = wbuf[pl.ds(g * 16, 16)]
            for e in range(16):
                i = g * 16 + e
                wv = jnp.full((16,), wv16[e], jnp.float32)
                for f in range(D // 16):
                    sl = pl.ds(f * 16, 16)
                    rows[i, sl] = rows[i, sl] * wv
            return 0
        lax.fori_loop(0, CHUNK // 16, _scale, 0)

        pltpu.sync_copy(rows, agg.at[idx_dst], add=True)
        return 0
    lax.fori_loop(0, NCHUNKS, _chunk, 0)
    plsc.subcore_barrier()

    for o, n in segs:
        pltpu.sync_copy(agg.at[pl.ds(r0 + o, n)], rows.at[pl.ds(0, n)])
        pltpu.sync_copy(rows.at[pl.ds(0, n)], out_hbm.at[c, pl.ds(r0 + o, n)])


_sc_agg = functools.partial(
    pl.kernel,
    out_type=jax.ShapeDtypeStruct((NC, N_PAD, D), jnp.float32),
    mesh=plsc.VectorSubcoreMesh(core_axis_name="c", subcore_axis_name="s",
                                num_cores=NC, num_subcores=NS),
    scratch_types=[
        pltpu.VMEM((CHUNK,), jnp.int32),
        pltpu.VMEM((CHUNK,), jnp.int32),
        pltpu.VMEM((CHUNK,), jnp.float32),
        pltpu.VMEM((CHUNK, D), jnp.float32),
        pltpu.VMEM_SHARED((N_PAD, D), jnp.float32),
        pltpu.SemaphoreType.DMA,
    ],
)(_sc_body)


def _tc_body(scal_ref, p_ref, h_ref, w_ref, o_ref):
    alpha = scal_ref[0]
    beta = scal_ref[1]
    left = (1.0 - alpha) * (p_ref[0] + p_ref[1]) + alpha * h_ref[...]
    o_ref[...] = (1.0 - beta) * left + beta * jnp.dot(
        left, w_ref[...], preferred_element_type=jnp.float32)


_ROWS_BLK = 1000

_tc_finish = pl.pallas_call(
    _tc_body,
    grid=(N // _ROWS_BLK,),
    in_specs=[
        pl.BlockSpec(memory_space=pltpu.SMEM),
        pl.BlockSpec((NC, _ROWS_BLK, D), lambda i: (0, i, 0)),
        pl.BlockSpec((_ROWS_BLK, D), lambda i: (i, 0)),
        pl.BlockSpec((D, D), lambda i: (0, 0)),
    ],
    out_specs=pl.BlockSpec((_ROWS_BLK, D), lambda i: (i, 0)),
    out_shape=jax.ShapeDtypeStruct((N, D), jnp.float32),
)


def kernel(x, h0, W, adj_values, adj_edge_index, alpha, beta):
    dst = adj_edge_index[0]
    src = adj_edge_index[1]
    partials = _sc_agg(x, src, dst, adj_values)
    scal = jnp.stack([jnp.asarray(alpha, jnp.float32),
                      jnp.asarray(beta, jnp.float32)])
    return _tc_finish(scal, partials, h0, W)


# speedup vs baseline: 4.5796x; 1.0122x over previous
import functools

import jax
import jax.numpy as jnp
from jax import lax
from jax.experimental import pallas as pl
from jax.experimental.pallas import tpu as pltpu
from jax.experimental.pallas import tpu_sc as plsc

N = 10000
E = 320000
D = 128

NC = 2
NS = 16
CHUNK = 64
CHUNKS_PER_TILE = 160
PASS_CHUNKS = 32
NPASS = CHUNKS_PER_TILE // PASS_CHUNKS
EP = NC * NS * CHUNKS_PER_TILE * CHUNK
EROWS = EP // CHUNK
N_PAD = 10240
ROWS_PER_TILE = N_PAD // NS
NZCOPY = ROWS_PER_TILE // CHUNK


def _sc_body(x_hbm, src_hbm, dst_hbm, w_hbm, out_hbm,
             esrc, edst, ew, r0b, r1b, r2b, r3b, agg,
             g0, g1, g2, g3, s0, s1, s2, s3):
    c = lax.axis_index("c")
    s = lax.axis_index("s")
    rows = (r0b, r1b, r2b, r3b)
    gsem = (g0, g1, g2, g3)
    ssem = (s0, s1, s2, s3)

    def _zrow(i, _):
        for f in range(D // 16):
            r0b[i, pl.ds(f * 16, 16)] = jnp.zeros((16,), jnp.float32)
        return 0
    lax.fori_loop(0, CHUNK, _zrow, 0)
    zbase = s * ROWS_PER_TILE
    for q in range(NZCOPY):
        pltpu.sync_copy(r0b, agg.at[pl.ds(zbase + q * CHUNK, CHUNK)])
    plsc.subcore_barrier()

    tile_row0 = (c * NS + s) * CHUNKS_PER_TILE

    def _gather(ch, u):
        return pltpu.async_copy(x_hbm.at[esrc.at[ch]], rows[u % 4],
                                gsem[u % 4])

    def _scatter(ch, u):
        return pltpu.async_copy(rows[u % 4], agg.at[edst.at[ch]],
                                ssem[u % 4], add=True)

    def _scatter_wait(ch, u):
        pltpu.make_async_copy(rows[u % 4], agg.at[edst.at[ch]],
                              ssem[u % 4]).wait()

    def _scale(ch, u):
        buf = rows[u % 4]

        def g_body(g, _):
            wv16 = ew[ch, pl.ds(g * 16, 16)]
            for e in range(16):
                wv = jnp.full((16,), wv16[e], jnp.float32)
                for f in range(D // 16):
                    sl = pl.ds(f * 16, 16)
                    buf[g * 16 + e, sl] = buf[g * 16 + e, sl] * wv
            return 0
        lax.fori_loop(0, CHUNK // 16, g_body, 0)

    def _pass(p, _):
        prow = tile_row0 + p * PASS_CHUNKS
        pltpu.sync_copy(src_hbm.at[pl.ds(prow, PASS_CHUNKS)], esrc)
        pltpu.sync_copy(dst_hbm.at[pl.ds(prow, PASS_CHUNKS)], edst)
        pltpu.sync_copy(w_hbm.at[pl.ds(prow, PASS_CHUNKS)], ew)

        _gather(0, 0)
        _gather(1, 1)

        def _quad(j, _):
            for u in range(4):
                ch = 4 * j + u
                if u in (0, 1):
                    @pl.when(j >= 1)
                    def _():
                        _scatter_wait(ch - 2, u + 2)
                    _gather(ch + 2, u + 2)
                else:
                    @pl.when(j < PASS_CHUNKS // 4 - 1)
                    def _():
                        _scatter_wait(ch - 2, u + 2)
                        _gather(ch + 2, u + 2)
                pltpu.make_async_copy(x_hbm.at[esrc.at[ch]], rows[u],
                                      gsem[u]).wait()
                _scale(ch, u)
                _scatter(ch, u)
            return 0
        lax.fori_loop(0, PASS_CHUNKS // 4, _quad, 0)
        for u in range(4):
            _scatter_wait(PASS_CHUNKS - 4 + u, u)
        return 0
    lax.fori_loop(0, NPASS, _pass, 0)
    plsc.subcore_barrier()

    for q in range(NZCOPY):
        o = zbase + q * CHUNK
        pltpu.sync_copy(agg.at[pl.ds(o, CHUNK)], r0b)
        pltpu.sync_copy(r0b, out_hbm.at[c, pl.ds(o, CHUNK)])


_sc_agg = functools.partial(
    pl.kernel,
    out_type=jax.ShapeDtypeStruct((NC, N_PAD, D), jnp.float32),
    mesh=plsc.VectorSubcoreMesh(core_axis_name="c", subcore_axis_name="s",
                                num_cores=NC, num_subcores=NS),
    scratch_types=[
        pltpu.VMEM((PASS_CHUNKS, CHUNK), jnp.int32),
        pltpu.VMEM((PASS_CHUNKS, CHUNK), jnp.int32),
        pltpu.VMEM((PASS_CHUNKS, CHUNK), jnp.float32),
        pltpu.VMEM((CHUNK, D), jnp.float32),
        pltpu.VMEM((CHUNK, D), jnp.float32),
        pltpu.VMEM((CHUNK, D), jnp.float32),
        pltpu.VMEM((CHUNK, D), jnp.float32),
        pltpu.VMEM_SHARED((N_PAD, D), jnp.float32),
        pltpu.SemaphoreType.DMA,
        pltpu.SemaphoreType.DMA,
        pltpu.SemaphoreType.DMA,
        pltpu.SemaphoreType.DMA,
        pltpu.SemaphoreType.DMA,
        pltpu.SemaphoreType.DMA,
        pltpu.SemaphoreType.DMA,
        pltpu.SemaphoreType.DMA,
    ],
)(_sc_body)


def _tc_body(scal_ref, p_ref, h_ref, w_ref, o_ref):
    alpha = scal_ref[0]
    beta = scal_ref[1]
    left = (1.0 - alpha) * (p_ref[0] + p_ref[1]) + alpha * h_ref[...]
    o_ref[...] = (1.0 - beta) * left + beta * jnp.dot(
        left, w_ref[...], preferred_element_type=jnp.float32)


_ROWS_BLK = 1000

_tc_finish = pl.pallas_call(
    _tc_body,
    grid=(N // _ROWS_BLK,),
    in_specs=[
        pl.BlockSpec(memory_space=pltpu.SMEM),
        pl.BlockSpec((NC, _ROWS_BLK, D), lambda i: (0, i, 0)),
        pl.BlockSpec((_ROWS_BLK, D), lambda i: (i, 0)),
        pl.BlockSpec((D, D), lambda i: (0, 0)),
    ],
    out_specs=pl.BlockSpec((_ROWS_BLK, D), lambda i: (i, 0)),
    out_shape=jax.ShapeDtypeStruct((N, D), jnp.float32),
)


def kernel(x, h0, W, adj_values, adj_edge_index, alpha, beta):
    dst = adj_edge_index[0]
    src = adj_edge_index[1]
    pad = EP - E
    srcp = jnp.concatenate([src, jnp.zeros((pad,), src.dtype)]).reshape(
        EROWS, CHUNK)
    dstp = jnp.concatenate([dst, jnp.zeros((pad,), dst.dtype)]).reshape(
        EROWS, CHUNK)
    wp = jnp.concatenate(
        [adj_values, jnp.zeros((pad,), adj_values.dtype)]).reshape(
        EROWS, CHUNK)
    partials = _sc_agg(x, srcp, dstp, wp)
    scal = jnp.stack([jnp.asarray(alpha, jnp.float32),
                      jnp.asarray(beta, jnp.float32)])
    return _tc_finish(scal, partials, h0, W)
